# Initial kernel scaffold; baseline (speedup 1.0000x reference)
import jax
import jax.numpy as jnp
from jax.experimental import pallas as pl


def kernel(embeddings, pos_table):
    raise NotImplementedError("write your pallas kernel here")


# speedup vs baseline: 1.4544x; 1.4544x over previous
import jax
import jax.numpy as jnp
from jax.experimental import pallas as pl


BATCH = 4
SEQ = 4096
DIM = 1024
BS = 512


def _body(emb_ref, pos_ref, out_ref):
    out_ref[...] = emb_ref[...] + pos_ref[...][None]


def kernel(embeddings, pos_table):
    b, s, d = embeddings.shape
    grid = (b, s // BS)
    return pl.pallas_call(
        _body,
        grid=grid,
        in_specs=[
            pl.BlockSpec((1, BS, d), lambda i, j: (i, j, 0)),
            pl.BlockSpec((BS, d), lambda i, j: (j, 0)),
        ],
        out_specs=pl.BlockSpec((1, BS, d), lambda i, j: (i, j, 0)),
        out_shape=jax.ShapeDtypeStruct((b, s, d), embeddings.dtype),
    )(embeddings, pos_table[:s])
